# baseline (speedup 1.0000x reference)
import functools

import jax
import jax.numpy as jnp
from jax import lax
from jax.experimental import pallas as pl
from jax.experimental.pallas import tpu as pltpu
from jax.experimental.pallas import tpu_sc as plsc

EPS = 1e-5

NC = 2
NS = 16
NW = NC * NS
CHUNK = 40


def _pack_bf16(h):
    K = h.shape[1] // 2
    h16 = h.astype(jnp.bfloat16)
    lo = lax.bitcast_convert_type(h16[:, :K], jnp.uint16).astype(jnp.uint32)
    hi = lax.bitcast_convert_type(h16[:, K:], jnp.uint16).astype(jnp.uint32)
    return lax.bitcast_convert_type(lo | (hi << 16), jnp.int32)


def _unpack_bf16(w):
    f_lo = lax.bitcast_convert_type(lax.shift_left(w, 16), jnp.float32)
    f_hi = lax.bitcast_convert_type(w & jnp.int32(-65536), jnp.float32)
    return jnp.concatenate([f_lo, f_hi], axis=1)


def _proj_body(x_ref, wsT_ref, bs_ref, wdT_ref, bd_ref, hs_ref, hd_ref):
    x = x_ref[...]
    hs_ref[...] = _pack_bf16(
        jnp.dot(x, wsT_ref[...], preferred_element_type=jnp.float32) + bs_ref[...]
    )
    hd_ref[...] = _pack_bf16(
        jnp.dot(x, wdT_ref[...], preferred_element_type=jnp.float32) + bd_ref[...]
    )


def _project(x, wsT, bs, wdT, bd):
    N, F = x.shape
    F2 = wsT.shape[1]
    BN = 2000 if N % 2000 == 0 else N
    return pl.pallas_call(
        _proj_body,
        grid=(N // BN,),
        in_specs=[
            pl.BlockSpec((BN, F), lambda i: (i, 0)),
            pl.BlockSpec((F, F2), lambda i: (0, 0)),
            pl.BlockSpec((1, F2), lambda i: (0, 0)),
            pl.BlockSpec((F, F2), lambda i: (0, 0)),
            pl.BlockSpec((1, F2), lambda i: (0, 0)),
        ],
        out_specs=[
            pl.BlockSpec((BN, F2 // 2), lambda i: (i, 0)),
            pl.BlockSpec((BN, F2 // 2), lambda i: (i, 0)),
        ],
        out_shape=[jax.ShapeDtypeStruct((N, F2 // 2), jnp.int32)] * 2,
    )(x, wsT, bs, wdT, bd)


def _gather_add(h_src, h_dst, src2, dst2):
    N, F2W = h_src.shape
    F2 = 2 * F2W
    EPW = src2.shape[1]
    E = NW * EPW
    NCH = EPW // CHUNK
    mesh = plsc.VectorSubcoreMesh(core_axis_name="c", subcore_axis_name="s")

    @functools.partial(
        pl.kernel,
        mesh=mesh,
        out_type=jax.ShapeDtypeStruct((E, F2), jnp.float32),
        scratch_types=[
            pltpu.VMEM((EPW,), jnp.int32),
            pltpu.VMEM((EPW,), jnp.int32),
            [pltpu.VMEM((CHUNK, F2W), jnp.int32) for _ in range(2)],
            [pltpu.VMEM((CHUNK, F2W), jnp.int32) for _ in range(2)],
            [pltpu.VMEM((CHUNK, F2), jnp.float32) for _ in range(2)],
            [pltpu.SemaphoreType.DMA for _ in range(2)],
            [pltpu.SemaphoreType.DMA for _ in range(2)],
            [pltpu.SemaphoreType.DMA for _ in range(2)],
        ],
    )
    def body(hs_hbm, hd_hbm, src_hbm, dst_hbm, g_hbm, ia, ib, ba, bb, bo, sa, sb, sw):
        c = lax.axis_index("c")
        s = lax.axis_index("s")
        wid = c * NS + s
        base0 = wid * EPW
        pltpu.sync_copy(src_hbm.at[wid], ia)
        pltpu.sync_copy(dst_hbm.at[wid], ib)

        def issue(i, p):
            sl = pl.ds(i * CHUNK, CHUNK)
            pltpu.async_copy(hs_hbm.at[ia.at[sl]], ba[p], sa[p])
            pltpu.async_copy(hd_hbm.at[ib.at[sl]], bb[p], sb[p])

        def wait_gather(p):
            pltpu.make_async_copy(hs_hbm.at[pl.ds(0, CHUNK)], ba[p], sa[p]).wait()
            pltpu.make_async_copy(hd_hbm.at[pl.ds(0, CHUNK)], bb[p], sb[p]).wait()

        def work(i, p, k):
            wait_gather(p)

            @pl.when(k > 0)
            def _drain():
                pltpu.make_async_copy(
                    g_hbm.at[pl.ds(0, CHUNK)], bo[p], sw[p]
                ).wait()

            def row(r, carry2):
                for kk in range(F2W // 16):
                    sl = pl.ds(kk * 16, 16)
                    wa = ba[p][r, sl]
                    wb = bb[p][r, sl]
                    lo = lax.bitcast_convert_type(
                        lax.shift_left(wa, 16), jnp.float32
                    ) + lax.bitcast_convert_type(lax.shift_left(wb, 16), jnp.float32)
                    hi = lax.bitcast_convert_type(
                        wa & jnp.int32(-65536), jnp.float32
                    ) + lax.bitcast_convert_type(wb & jnp.int32(-65536), jnp.float32)
                    bo[p][r, pl.ds(kk * 16, 16)] = lo
                    bo[p][r, pl.ds(F2W + kk * 16, 16)] = hi
                return carry2

            lax.fori_loop(0, CHUNK, row, 0)
            pltpu.async_copy(bo[p], g_hbm.at[pl.ds(base0 + i * CHUNK, CHUNK)], sw[p])

        issue(0, 0)
        issue(1, 1)

        def pair(k, carry):
            i0 = 2 * k
            work(i0, 0, k)

            @pl.when(i0 + 2 < NCH)
            def _i0():
                issue(i0 + 2, 0)

            work(i0 + 1, 1, k)

            @pl.when(i0 + 3 < NCH)
            def _i1():
                issue(i0 + 3, 1)

            return carry

        lax.fori_loop(0, NCH // 2, pair, 0)
        pltpu.make_async_copy(g_hbm.at[pl.ds(0, CHUNK)], bo[0], sw[0]).wait()
        pltpu.make_async_copy(g_hbm.at[pl.ds(0, CHUNK)], bo[1], sw[1]).wait()

    return body(h_src, h_dst, src2, dst2)


def _stats_body(g_ref, ef_ref, weT_ref, be_ref, out_ref, acc_ref):
    i = pl.program_id(0)

    @pl.when(i == 0)
    def _init():
        acc_ref[...] = jnp.zeros_like(acc_ref)

    m = (
        g_ref[...]
        + jnp.dot(ef_ref[...], weT_ref[...], preferred_element_type=jnp.float32)
        + be_ref[...]
    )
    acc_ref[0:1, :] += jnp.sum(m, axis=0, keepdims=True)
    acc_ref[1:2, :] += jnp.sum(m * m, axis=0, keepdims=True)

    @pl.when(i == pl.num_programs(0) - 1)
    def _fin():
        out_ref[...] = acc_ref[...]


def _edge_stats(g, ef, weT, be):
    E, F2 = g.shape
    FE = ef.shape[1]
    BE = 4000 if E % 4000 == 0 else E
    return pl.pallas_call(
        _stats_body,
        grid=(E // BE,),
        in_specs=[
            pl.BlockSpec((BE, F2), lambda i: (i, 0)),
            pl.BlockSpec((BE, FE), lambda i: (i, 0)),
            pl.BlockSpec((FE, F2), lambda i: (0, 0)),
            pl.BlockSpec((1, F2), lambda i: (0, 0)),
        ],
        out_specs=pl.BlockSpec((2, F2), lambda i: (0, 0)),
        out_shape=jax.ShapeDtypeStruct((2, F2), jnp.float32),
        scratch_shapes=[pltpu.VMEM((2, F2), jnp.float32)],
    )(g, ef, weT, be)


def _act_body(E, F, g_ref, ef_ref, weT_ref, be_ref, st_ref, gm_ref, bm_ref, y_ref):
    m = (
        g_ref[...]
        + jnp.dot(ef_ref[...], weT_ref[...], preferred_element_type=jnp.float32)
        + be_ref[...]
    )
    mu = st_ref[0:1, :] / E
    var = st_ref[1:2, :] / E - mu * mu
    scale = gm_ref[...] * lax.rsqrt(var + EPS)
    shift = bm_ref[...] - mu * scale
    mn = m * scale + shift
    f = mn[:, :F]
    sp = mn[:, F:]
    y_ref[...] = jax.nn.sigmoid(f) * jax.nn.softplus(sp)


def _edge_activate(g, ef, weT, be, stats, gm, bm):
    E, F2 = g.shape
    F = F2 // 2
    FE = ef.shape[1]
    BE = 4000 if E % 4000 == 0 else E
    return pl.pallas_call(
        functools.partial(_act_body, E, F),
        grid=(E // BE,),
        in_specs=[
            pl.BlockSpec((BE, F2), lambda i: (i, 0)),
            pl.BlockSpec((BE, FE), lambda i: (i, 0)),
            pl.BlockSpec((FE, F2), lambda i: (0, 0)),
            pl.BlockSpec((1, F2), lambda i: (0, 0)),
            pl.BlockSpec((2, F2), lambda i: (0, 0)),
            pl.BlockSpec((1, F2), lambda i: (0, 0)),
            pl.BlockSpec((1, F2), lambda i: (0, 0)),
        ],
        out_specs=pl.BlockSpec((BE, F), lambda i: (i, 0)),
        out_shape=jax.ShapeDtypeStruct((E, F), jnp.float32),
    )(g, ef, weT, be, stats, gm, bm)


def _scatter_sum(y, dst3, N):
    E, F = y.shape
    EPW = E // NW
    NCH = EPW // CHUNK
    NDW = 10
    RPT = N // NDW
    ZB = 40
    mesh = plsc.VectorSubcoreMesh(core_axis_name="c", subcore_axis_name="s")

    @functools.partial(
        pl.kernel,
        mesh=mesh,
        out_type=jax.ShapeDtypeStruct((NC, N, F), jnp.float32),
        scratch_types=[
            pltpu.VMEM((NCH, CHUNK), jnp.int32),
            [pltpu.VMEM((CHUNK, F), jnp.float32) for _ in range(2)],
            pltpu.VMEM((ZB, F), jnp.float32),
            pltpu.VMEM_SHARED((N, F), jnp.float32),
            [pltpu.SemaphoreType.DMA for _ in range(2)],
            [pltpu.SemaphoreType.DMA for _ in range(2)],
        ],
    )
    def body(y_hbm, dst_hbm, out_hbm, idx, yb, zb, acc, sy, sc_sem):
        c = lax.axis_index("c")
        s = lax.axis_index("s")
        wid = c * NS + s
        pltpu.sync_copy(dst_hbm.at[wid], idx)

        def zrow(r, carry):
            for k in range(F // 16):
                zb[r, pl.ds(k * 16, 16)] = jnp.zeros((16,), jnp.float32)
            return carry

        lax.fori_loop(0, ZB, zrow, 0)

        @pl.when(s < NDW)
        def _zero():
            def zchunk(j, carry):
                pltpu.sync_copy(zb, acc.at[pl.ds(s * RPT + j * ZB, ZB)])
                return carry

            lax.fori_loop(0, RPT // ZB, zchunk, 0)

        plsc.subcore_barrier()

        base0 = wid * EPW

        def issue(i, p):
            pltpu.async_copy(y_hbm.at[pl.ds(base0 + i * CHUNK, CHUNK)], yb[p], sy[p])

        def work(i, p):
            pltpu.make_async_copy(y_hbm.at[pl.ds(0, CHUNK)], yb[p], sy[p]).wait()
            pltpu.async_copy(yb[p], acc.at[idx.at[i]], sc_sem[p], add=True)

        def refill(i, p):
            pltpu.make_async_copy(yb[p], acc.at[pl.ds(0, CHUNK)], sc_sem[p]).wait()
            issue(i, p)

        issue(0, 0)
        issue(1, 1)

        def pair(k, carry):
            i0 = 2 * k
            work(i0, 0)

            @pl.when(i0 + 2 < NCH)
            def _i0():
                refill(i0 + 2, 0)

            work(i0 + 1, 1)

            @pl.when(i0 + 3 < NCH)
            def _i1():
                refill(i0 + 3, 1)

            return carry

        lax.fori_loop(0, NCH // 2, pair, 0)
        pltpu.make_async_copy(yb[0], acc.at[pl.ds(0, CHUNK)], sc_sem[0]).wait()
        pltpu.make_async_copy(yb[1], acc.at[pl.ds(0, CHUNK)], sc_sem[1]).wait()
        plsc.subcore_barrier()

        @pl.when(s < NDW)
        def _dump():
            pltpu.sync_copy(
                acc.at[pl.ds(s * RPT, RPT)], out_hbm.at[c, pl.ds(s * RPT, RPT)]
            )

    return body(y, dst3)


def _final_body(p_ref, x_ref, gamma_ref, beta_ref, out_ref):
    h = p_ref[0] + p_ref[1]
    N = h.shape[0]
    mu = jnp.sum(h, axis=0, keepdims=True) / N
    var = jnp.sum(h * h, axis=0, keepdims=True) / N - mu * mu
    hn = (h - mu) * lax.rsqrt(var + EPS) * gamma_ref[...] + beta_ref[...]
    out_ref[...] = jax.nn.softplus(x_ref[...] + hn)


def _final(partials, x, gamma, beta):
    N, F = x.shape
    return pl.pallas_call(
        _final_body,
        out_shape=jax.ShapeDtypeStruct((N, F), jnp.float32),
    )(partials, x, gamma, beta)


def kernel(node_feats, edge_feats, edge_index, W_src, b_src, W_dst, b_dst,
           W_edge, b_edge, gamma_m, beta_m, gamma, beta):
    N, F = node_feats.shape
    E = edge_feats.shape[0]
    EPW = E // NW
    NCH = EPW // CHUNK
    src2 = edge_index[0].reshape(NW, EPW)
    dst2 = edge_index[1].reshape(NW, EPW)
    dst3 = edge_index[1].reshape(NW, NCH, CHUNK)

    h_src, h_dst = _project(
        node_feats, W_src.T, b_src.reshape(1, -1), W_dst.T, b_dst.reshape(1, -1)
    )
    g = _gather_add(h_src, h_dst, src2, dst2)
    stats = _edge_stats(g, edge_feats, W_edge.T, b_edge.reshape(1, -1))
    y = _edge_activate(
        g, edge_feats, W_edge.T, b_edge.reshape(1, -1), stats,
        gamma_m.reshape(1, -1), beta_m.reshape(1, -1),
    )
    partials = _scatter_sum(y, dst3, N)
    out = _final(partials, node_feats, gamma.reshape(1, -1), beta.reshape(1, -1))
    return out


# speedup vs baseline: 1.1970x; 1.1970x over previous
import functools

import jax
import jax.numpy as jnp
from jax import lax
from jax.experimental import pallas as pl
from jax.experimental.pallas import tpu as pltpu
from jax.experimental.pallas import tpu_sc as plsc

EPS = 1e-5

NC = 2
NS = 16
NW = NC * NS
CHUNK = 40


def _pack_bf16(h):
    K = h.shape[1] // 2
    h16 = h.astype(jnp.bfloat16)
    lo = lax.bitcast_convert_type(h16[:, :K], jnp.uint16).astype(jnp.uint32)
    hi = lax.bitcast_convert_type(h16[:, K:], jnp.uint16).astype(jnp.uint32)
    return lax.bitcast_convert_type(lo | (hi << 16), jnp.int32)


def _unpack_bf16(w):
    f_lo = lax.bitcast_convert_type(lax.shift_left(w, 16), jnp.float32)
    f_hi = lax.bitcast_convert_type(w & jnp.int32(-65536), jnp.float32)
    return jnp.concatenate([f_lo, f_hi], axis=1)


def _proj_body(x_ref, wsT_ref, bs_ref, wdT_ref, bd_ref, hs_ref, hd_ref):
    x = x_ref[...]
    hs_ref[...] = _pack_bf16(
        jnp.dot(x, wsT_ref[...], preferred_element_type=jnp.float32) + bs_ref[...]
    )
    hd_ref[...] = _pack_bf16(
        jnp.dot(x, wdT_ref[...], preferred_element_type=jnp.float32) + bd_ref[...]
    )


def _project(x, wsT, bs, wdT, bd):
    N, F = x.shape
    F2 = wsT.shape[1]
    BN = 2000 if N % 2000 == 0 else N
    return pl.pallas_call(
        _proj_body,
        grid=(N // BN,),
        in_specs=[
            pl.BlockSpec((BN, F), lambda i: (i, 0)),
            pl.BlockSpec((F, F2), lambda i: (0, 0)),
            pl.BlockSpec((1, F2), lambda i: (0, 0)),
            pl.BlockSpec((F, F2), lambda i: (0, 0)),
            pl.BlockSpec((1, F2), lambda i: (0, 0)),
        ],
        out_specs=[
            pl.BlockSpec((BN, F2 // 2), lambda i: (i, 0)),
            pl.BlockSpec((BN, F2 // 2), lambda i: (i, 0)),
        ],
        out_shape=[jax.ShapeDtypeStruct((N, F2 // 2), jnp.int32)] * 2,
    )(x, wsT, bs, wdT, bd)


def _gather_pair(h_src, h_dst, src2, dst2):
    N, F2W = h_src.shape
    EPW = src2.shape[1]
    E = NW * EPW
    NCH = EPW // CHUNK
    mesh = plsc.VectorSubcoreMesh(core_axis_name="c", subcore_axis_name="s")

    @functools.partial(
        pl.kernel,
        mesh=mesh,
        out_type=[jax.ShapeDtypeStruct((E, F2W), jnp.int32)] * 2,
        scratch_types=[
            pltpu.VMEM((EPW,), jnp.int32),
            pltpu.VMEM((EPW,), jnp.int32),
            [pltpu.VMEM((CHUNK, F2W), jnp.int32) for _ in range(2)],
            [pltpu.VMEM((CHUNK, F2W), jnp.int32) for _ in range(2)],
            [pltpu.SemaphoreType.DMA for _ in range(2)],
            [pltpu.SemaphoreType.DMA for _ in range(2)],
            [pltpu.SemaphoreType.DMA for _ in range(2)],
            [pltpu.SemaphoreType.DMA for _ in range(2)],
        ],
    )
    def body(hs_hbm, hd_hbm, src_hbm, dst_hbm, ga_hbm, gb_hbm,
             ia, ib, ba, bb, sa, sb, swa, swb):
        c = lax.axis_index("c")
        s = lax.axis_index("s")
        wid = c * NS + s
        base0 = wid * EPW
        pltpu.sync_copy(src_hbm.at[wid], ia)
        pltpu.sync_copy(dst_hbm.at[wid], ib)

        def issue(i, p):
            sl = pl.ds(i * CHUNK, CHUNK)
            pltpu.async_copy(hs_hbm.at[ia.at[sl]], ba[p], sa[p])
            pltpu.async_copy(hd_hbm.at[ib.at[sl]], bb[p], sb[p])

        def work(i, p):
            pltpu.make_async_copy(hs_hbm.at[pl.ds(0, CHUNK)], ba[p], sa[p]).wait()
            pltpu.make_async_copy(hd_hbm.at[pl.ds(0, CHUNK)], bb[p], sb[p]).wait()
            osl = pl.ds(base0 + i * CHUNK, CHUNK)
            pltpu.async_copy(ba[p], ga_hbm.at[osl], swa[p])
            pltpu.async_copy(bb[p], gb_hbm.at[osl], swb[p])

        def refill(i, p):
            pltpu.make_async_copy(ga_hbm.at[pl.ds(0, CHUNK)], ba[p], swa[p]).wait()
            pltpu.make_async_copy(gb_hbm.at[pl.ds(0, CHUNK)], bb[p], swb[p]).wait()
            issue(i, p)

        issue(0, 0)
        issue(1, 1)

        def pair(k, carry):
            i0 = 2 * k
            work(i0, 0)

            @pl.when(i0 + 2 < NCH)
            def _i0():
                refill(i0 + 2, 0)

            work(i0 + 1, 1)

            @pl.when(i0 + 3 < NCH)
            def _i1():
                refill(i0 + 3, 1)

            return carry

        lax.fori_loop(0, NCH // 2, pair, 0)
        for p in range(2):
            pltpu.make_async_copy(ga_hbm.at[pl.ds(0, CHUNK)], ba[p], swa[p]).wait()
            pltpu.make_async_copy(gb_hbm.at[pl.ds(0, CHUNK)], bb[p], swb[p]).wait()

    return body(h_src, h_dst, src2, dst2)


def _stats_body(ga_ref, gb_ref, ef_ref, weT_ref, be_ref, out_ref, acc_ref):
    i = pl.program_id(0)

    @pl.when(i == 0)
    def _init():
        acc_ref[...] = jnp.zeros_like(acc_ref)

    m = (
        _unpack_bf16(ga_ref[...])
        + _unpack_bf16(gb_ref[...])
        + jnp.dot(ef_ref[...], weT_ref[...], preferred_element_type=jnp.float32)
        + be_ref[...]
    )
    acc_ref[0:1, :] += jnp.sum(m, axis=0, keepdims=True)
    acc_ref[1:2, :] += jnp.sum(m * m, axis=0, keepdims=True)

    @pl.when(i == pl.num_programs(0) - 1)
    def _fin():
        out_ref[...] = acc_ref[...]


def _edge_stats(ga, gb, ef, weT, be):
    E, F2W = ga.shape
    F2 = 2 * F2W
    FE = ef.shape[1]
    BE = 4000 if E % 4000 == 0 else E
    return pl.pallas_call(
        _stats_body,
        grid=(E // BE,),
        in_specs=[
            pl.BlockSpec((BE, F2W), lambda i: (i, 0)),
            pl.BlockSpec((BE, F2W), lambda i: (i, 0)),
            pl.BlockSpec((BE, FE), lambda i: (i, 0)),
            pl.BlockSpec((FE, F2), lambda i: (0, 0)),
            pl.BlockSpec((1, F2), lambda i: (0, 0)),
        ],
        out_specs=pl.BlockSpec((2, F2), lambda i: (0, 0)),
        out_shape=jax.ShapeDtypeStruct((2, F2), jnp.float32),
        scratch_shapes=[pltpu.VMEM((2, F2), jnp.float32)],
    )(ga, gb, ef, weT, be)


def _act_body(E, F, ga_ref, gb_ref, ef_ref, weT_ref, be_ref, st_ref, gm_ref,
              bm_ref, y_ref):
    m = (
        _unpack_bf16(ga_ref[...])
        + _unpack_bf16(gb_ref[...])
        + jnp.dot(ef_ref[...], weT_ref[...], preferred_element_type=jnp.float32)
        + be_ref[...]
    )
    mu = st_ref[0:1, :] / E
    var = st_ref[1:2, :] / E - mu * mu
    scale = gm_ref[...] * lax.rsqrt(var + EPS)
    shift = bm_ref[...] - mu * scale
    mn = m * scale + shift
    f = mn[:, :F]
    sp = mn[:, F:]
    y_ref[...] = jax.nn.sigmoid(f) * jax.nn.softplus(sp)


def _edge_activate(ga, gb, ef, weT, be, stats, gm, bm):
    E, F2W = ga.shape
    F2 = 2 * F2W
    F = F2 // 2
    FE = ef.shape[1]
    BE = 4000 if E % 4000 == 0 else E
    return pl.pallas_call(
        functools.partial(_act_body, E, F),
        grid=(E // BE,),
        in_specs=[
            pl.BlockSpec((BE, F2W), lambda i: (i, 0)),
            pl.BlockSpec((BE, F2W), lambda i: (i, 0)),
            pl.BlockSpec((BE, FE), lambda i: (i, 0)),
            pl.BlockSpec((FE, F2), lambda i: (0, 0)),
            pl.BlockSpec((1, F2), lambda i: (0, 0)),
            pl.BlockSpec((2, F2), lambda i: (0, 0)),
            pl.BlockSpec((1, F2), lambda i: (0, 0)),
            pl.BlockSpec((1, F2), lambda i: (0, 0)),
        ],
        out_specs=pl.BlockSpec((BE, F), lambda i: (i, 0)),
        out_shape=jax.ShapeDtypeStruct((E, F), jnp.float32),
    )(ga, gb, ef, weT, be, stats, gm, bm)


def _scatter_sum(y, dst3, N):
    E, F = y.shape
    EPW = E // NW
    NCH = EPW // CHUNK
    NDW = 10
    RPT = N // NDW
    ZB = 40
    mesh = plsc.VectorSubcoreMesh(core_axis_name="c", subcore_axis_name="s")

    @functools.partial(
        pl.kernel,
        mesh=mesh,
        out_type=jax.ShapeDtypeStruct((NC, N, F), jnp.float32),
        scratch_types=[
            pltpu.VMEM((NCH, CHUNK), jnp.int32),
            [pltpu.VMEM((CHUNK, F), jnp.float32) for _ in range(2)],
            pltpu.VMEM((ZB, F), jnp.float32),
            pltpu.VMEM_SHARED((N, F), jnp.float32),
            [pltpu.SemaphoreType.DMA for _ in range(2)],
            [pltpu.SemaphoreType.DMA for _ in range(2)],
        ],
    )
    def body(y_hbm, dst_hbm, out_hbm, idx, yb, zb, acc, sy, sc_sem):
        c = lax.axis_index("c")
        s = lax.axis_index("s")
        wid = c * NS + s
        pltpu.sync_copy(dst_hbm.at[wid], idx)

        def zrow(r, carry):
            for k in range(F // 16):
                zb[r, pl.ds(k * 16, 16)] = jnp.zeros((16,), jnp.float32)
            return carry

        lax.fori_loop(0, ZB, zrow, 0)

        @pl.when(s < NDW)
        def _zero():
            def zchunk(j, carry):
                pltpu.sync_copy(zb, acc.at[pl.ds(s * RPT + j * ZB, ZB)])
                return carry

            lax.fori_loop(0, RPT // ZB, zchunk, 0)

        plsc.subcore_barrier()

        base0 = wid * EPW

        def issue(i, p):
            pltpu.async_copy(y_hbm.at[pl.ds(base0 + i * CHUNK, CHUNK)], yb[p], sy[p])

        def work(i, p):
            pltpu.make_async_copy(y_hbm.at[pl.ds(0, CHUNK)], yb[p], sy[p]).wait()
            pltpu.async_copy(yb[p], acc.at[idx.at[i]], sc_sem[p], add=True)

        def refill(i, p):
            pltpu.make_async_copy(yb[p], acc.at[pl.ds(0, CHUNK)], sc_sem[p]).wait()
            issue(i, p)

        issue(0, 0)
        issue(1, 1)

        def pair(k, carry):
            i0 = 2 * k
            work(i0, 0)

            @pl.when(i0 + 2 < NCH)
            def _i0():
                refill(i0 + 2, 0)

            work(i0 + 1, 1)

            @pl.when(i0 + 3 < NCH)
            def _i1():
                refill(i0 + 3, 1)

            return carry

        lax.fori_loop(0, NCH // 2, pair, 0)
        pltpu.make_async_copy(yb[0], acc.at[pl.ds(0, CHUNK)], sc_sem[0]).wait()
        pltpu.make_async_copy(yb[1], acc.at[pl.ds(0, CHUNK)], sc_sem[1]).wait()
        plsc.subcore_barrier()

        @pl.when(s < NDW)
        def _dump():
            pltpu.sync_copy(
                acc.at[pl.ds(s * RPT, RPT)], out_hbm.at[c, pl.ds(s * RPT, RPT)]
            )

    return body(y, dst3)


def _final_body(p_ref, x_ref, gamma_ref, beta_ref, out_ref):
    h = p_ref[0] + p_ref[1]
    N = h.shape[0]
    mu = jnp.sum(h, axis=0, keepdims=True) / N
    var = jnp.sum(h * h, axis=0, keepdims=True) / N - mu * mu
    hn = (h - mu) * lax.rsqrt(var + EPS) * gamma_ref[...] + beta_ref[...]
    out_ref[...] = jax.nn.softplus(x_ref[...] + hn)


def _final(partials, x, gamma, beta):
    N, F = x.shape
    return pl.pallas_call(
        _final_body,
        out_shape=jax.ShapeDtypeStruct((N, F), jnp.float32),
    )(partials, x, gamma, beta)


def kernel(node_feats, edge_feats, edge_index, W_src, b_src, W_dst, b_dst,
           W_edge, b_edge, gamma_m, beta_m, gamma, beta):
    N, F = node_feats.shape
    E = edge_feats.shape[0]
    EPW = E // NW
    NCH = EPW // CHUNK
    src2 = edge_index[0].reshape(NW, EPW)
    dst2 = edge_index[1].reshape(NW, EPW)
    dst3 = edge_index[1].reshape(NW, NCH, CHUNK)

    h_src, h_dst = _project(
        node_feats, W_src.T, b_src.reshape(1, -1), W_dst.T, b_dst.reshape(1, -1)
    )
    ga, gb = _gather_pair(h_src, h_dst, src2, dst2)
    stats = _edge_stats(ga, gb, edge_feats, W_edge.T, b_edge.reshape(1, -1))
    y = _edge_activate(
        ga, gb, edge_feats, W_edge.T, b_edge.reshape(1, -1), stats,
        gamma_m.reshape(1, -1), beta_m.reshape(1, -1),
    )
    partials = _scatter_sum(y, dst3, N)
    out = _final(partials, node_feats, gamma.reshape(1, -1), beta.reshape(1, -1))
    return out


# speedup vs baseline: 1.3547x; 1.1317x over previous
import functools

import jax
import jax.numpy as jnp
from jax import lax
from jax.experimental import pallas as pl
from jax.experimental.pallas import tpu as pltpu
from jax.experimental.pallas import tpu_sc as plsc

EPS = 1e-5

NC = 2
NS = 16
NW = NC * NS
CHUNK = 40
NBUF = 5


def _pack_bf16(h):
    K = h.shape[1] // 2
    h16 = h.astype(jnp.bfloat16)
    lo = lax.bitcast_convert_type(h16[:, :K], jnp.uint16).astype(jnp.uint32)
    hi = lax.bitcast_convert_type(h16[:, K:], jnp.uint16).astype(jnp.uint32)
    return lax.bitcast_convert_type(lo | (hi << 16), jnp.int32)


def _unpack_bf16(w):
    f_lo = lax.bitcast_convert_type(lax.shift_left(w, 16), jnp.float32)
    f_hi = lax.bitcast_convert_type(w & jnp.int32(-65536), jnp.float32)
    return jnp.concatenate([f_lo, f_hi], axis=1)


def _proj_body(x_ref, wsT_ref, bs_ref, wdT_ref, bd_ref, hs_ref, hd_ref):
    x = x_ref[...]
    hs_ref[...] = _pack_bf16(
        jnp.dot(x, wsT_ref[...], preferred_element_type=jnp.float32) + bs_ref[...]
    )
    hd_ref[...] = _pack_bf16(
        jnp.dot(x, wdT_ref[...], preferred_element_type=jnp.float32) + bd_ref[...]
    )


def _project(x, wsT, bs, wdT, bd):
    N, F = x.shape
    F2 = wsT.shape[1]
    BN = 2000 if N % 2000 == 0 else N
    return pl.pallas_call(
        _proj_body,
        grid=(N // BN,),
        in_specs=[
            pl.BlockSpec((BN, F), lambda i: (i, 0)),
            pl.BlockSpec((F, F2), lambda i: (0, 0)),
            pl.BlockSpec((1, F2), lambda i: (0, 0)),
            pl.BlockSpec((F, F2), lambda i: (0, 0)),
            pl.BlockSpec((1, F2), lambda i: (0, 0)),
        ],
        out_specs=[
            pl.BlockSpec((BN, F2 // 2), lambda i: (i, 0)),
            pl.BlockSpec((BN, F2 // 2), lambda i: (i, 0)),
        ],
        out_shape=[jax.ShapeDtypeStruct((N, F2 // 2), jnp.int32)] * 2,
    )(x, wsT, bs, wdT, bd)


def _gather_pair(h_src, h_dst, src2, dst2):
    N, F2W = h_src.shape
    EPW = src2.shape[1]
    E = NW * EPW
    NCH = EPW // CHUNK
    mesh = plsc.VectorSubcoreMesh(core_axis_name="c", subcore_axis_name="s")

    @functools.partial(
        pl.kernel,
        mesh=mesh,
        out_type=[jax.ShapeDtypeStruct((E, F2W), jnp.int32)] * 2,
        scratch_types=[
            pltpu.VMEM((EPW,), jnp.int32),
            pltpu.VMEM((EPW,), jnp.int32),
            [pltpu.VMEM((CHUNK, F2W), jnp.int32) for _ in range(NBUF)],
            [pltpu.VMEM((CHUNK, F2W), jnp.int32) for _ in range(NBUF)],
            [pltpu.SemaphoreType.DMA for _ in range(NBUF)],
            [pltpu.SemaphoreType.DMA for _ in range(NBUF)],
            [pltpu.SemaphoreType.DMA for _ in range(NBUF)],
            [pltpu.SemaphoreType.DMA for _ in range(NBUF)],
        ],
    )
    def body(hs_hbm, hd_hbm, src_hbm, dst_hbm, ga_hbm, gb_hbm,
             ia, ib, ba, bb, sa, sb, swa, swb):
        c = lax.axis_index("c")
        s = lax.axis_index("s")
        wid = c * NS + s
        base0 = wid * EPW
        pltpu.sync_copy(src_hbm.at[wid], ia)
        pltpu.sync_copy(dst_hbm.at[wid], ib)

        def issue(i, p):
            sl = pl.ds(i * CHUNK, CHUNK)
            pltpu.async_copy(hs_hbm.at[ia.at[sl]], ba[p], sa[p])
            pltpu.async_copy(hd_hbm.at[ib.at[sl]], bb[p], sb[p])

        def work(i, p):
            pltpu.make_async_copy(hs_hbm.at[pl.ds(0, CHUNK)], ba[p], sa[p]).wait()
            pltpu.make_async_copy(hd_hbm.at[pl.ds(0, CHUNK)], bb[p], sb[p]).wait()
            osl = pl.ds(base0 + i * CHUNK, CHUNK)
            pltpu.async_copy(ba[p], ga_hbm.at[osl], swa[p])
            pltpu.async_copy(bb[p], gb_hbm.at[osl], swb[p])

        def refill(i, p):
            pltpu.make_async_copy(ga_hbm.at[pl.ds(0, CHUNK)], ba[p], swa[p]).wait()
            pltpu.make_async_copy(gb_hbm.at[pl.ds(0, CHUNK)], bb[p], swb[p]).wait()
            issue(i, p)

        for p in range(NBUF):
            issue(p, p)

        def group(k, carry):
            i0 = NBUF * k
            for p in range(NBUF):
                work(i0 + p, p)

                @pl.when(i0 + p + NBUF < NCH)
                def _r(p=p, i=i0 + p + NBUF):
                    refill(i, p)

            return carry

        lax.fori_loop(0, NCH // NBUF, group, 0)
        for p in range(NBUF):
            pltpu.make_async_copy(ga_hbm.at[pl.ds(0, CHUNK)], ba[p], swa[p]).wait()
            pltpu.make_async_copy(gb_hbm.at[pl.ds(0, CHUNK)], bb[p], swb[p]).wait()

    return body(h_src, h_dst, src2, dst2)


def _stats_body(ga_ref, gb_ref, ef_ref, weT_ref, be_ref, out_ref, acc_ref):
    i = pl.program_id(0)

    @pl.when(i == 0)
    def _init():
        acc_ref[...] = jnp.zeros_like(acc_ref)

    m = (
        _unpack_bf16(ga_ref[...])
        + _unpack_bf16(gb_ref[...])
        + jnp.dot(ef_ref[...], weT_ref[...], preferred_element_type=jnp.float32)
        + be_ref[...]
    )
    acc_ref[0:1, :] += jnp.sum(m, axis=0, keepdims=True)
    acc_ref[1:2, :] += jnp.sum(m * m, axis=0, keepdims=True)

    @pl.when(i == pl.num_programs(0) - 1)
    def _fin():
        out_ref[...] = acc_ref[...]


def _edge_stats(ga, gb, ef, weT, be):
    E, F2W = ga.shape
    F2 = 2 * F2W
    FE = ef.shape[1]
    BE = 8000 if E % 8000 == 0 else E
    return pl.pallas_call(
        _stats_body,
        grid=(E // BE,),
        in_specs=[
            pl.BlockSpec((BE, F2W), lambda i: (i, 0)),
            pl.BlockSpec((BE, F2W), lambda i: (i, 0)),
            pl.BlockSpec((BE, FE), lambda i: (i, 0)),
            pl.BlockSpec((FE, F2), lambda i: (0, 0)),
            pl.BlockSpec((1, F2), lambda i: (0, 0)),
        ],
        out_specs=pl.BlockSpec((2, F2), lambda i: (0, 0)),
        out_shape=jax.ShapeDtypeStruct((2, F2), jnp.float32),
        scratch_shapes=[pltpu.VMEM((2, F2), jnp.float32)],
    )(ga, gb, ef, weT, be)


def _act_body(E, F, ga_ref, gb_ref, ef_ref, weT_ref, be_ref, st_ref, gm_ref,
              bm_ref, y_ref):
    m = (
        _unpack_bf16(ga_ref[...])
        + _unpack_bf16(gb_ref[...])
        + jnp.dot(ef_ref[...], weT_ref[...], preferred_element_type=jnp.float32)
        + be_ref[...]
    )
    mu = st_ref[0:1, :] / E
    var = st_ref[1:2, :] / E - mu * mu
    scale = gm_ref[...] * lax.rsqrt(var + EPS)
    shift = bm_ref[...] - mu * scale
    mn = m * scale + shift
    f = mn[:, :F]
    sp = mn[:, F:]
    y_ref[...] = jax.nn.sigmoid(f) * jax.nn.softplus(sp)


def _edge_activate(ga, gb, ef, weT, be, stats, gm, bm):
    E, F2W = ga.shape
    F2 = 2 * F2W
    F = F2 // 2
    FE = ef.shape[1]
    BE = 8000 if E % 8000 == 0 else E
    return pl.pallas_call(
        functools.partial(_act_body, E, F),
        grid=(E // BE,),
        in_specs=[
            pl.BlockSpec((BE, F2W), lambda i: (i, 0)),
            pl.BlockSpec((BE, F2W), lambda i: (i, 0)),
            pl.BlockSpec((BE, FE), lambda i: (i, 0)),
            pl.BlockSpec((FE, F2), lambda i: (0, 0)),
            pl.BlockSpec((1, F2), lambda i: (0, 0)),
            pl.BlockSpec((2, F2), lambda i: (0, 0)),
            pl.BlockSpec((1, F2), lambda i: (0, 0)),
            pl.BlockSpec((1, F2), lambda i: (0, 0)),
        ],
        out_specs=pl.BlockSpec((BE, F), lambda i: (i, 0)),
        out_shape=jax.ShapeDtypeStruct((E, F), jnp.float32),
    )(ga, gb, ef, weT, be, stats, gm, bm)


def _scatter_sum(y, dst3, N):
    E, F = y.shape
    EPW = E // NW
    NCH = EPW // CHUNK
    NST, SCH = dst3.shape[1], dst3.shape[2]
    NDW = 10
    RPT = N // NDW
    ZB = 40
    mesh = plsc.VectorSubcoreMesh(core_axis_name="c", subcore_axis_name="s")

    @functools.partial(
        pl.kernel,
        mesh=mesh,
        out_type=jax.ShapeDtypeStruct((NC, N, F), jnp.float32),
        scratch_types=[
            pltpu.VMEM((SCH, CHUNK), jnp.int32),
            [pltpu.VMEM((CHUNK, F), jnp.float32) for _ in range(NBUF)],
            pltpu.VMEM((ZB, F), jnp.float32),
            pltpu.VMEM_SHARED((N, F), jnp.float32),
            [pltpu.SemaphoreType.DMA for _ in range(NBUF)],
            [pltpu.SemaphoreType.DMA for _ in range(NBUF)],
        ],
    )
    def body(y_hbm, dst_hbm, out_hbm, idx, yb, zb, acc, sy, sc_sem):
        c = lax.axis_index("c")
        s = lax.axis_index("s")
        wid = c * NS + s

        def zrow(r, carry):
            for k in range(F // 16):
                zb[r, pl.ds(k * 16, 16)] = jnp.zeros((16,), jnp.float32)
            return carry

        lax.fori_loop(0, ZB, zrow, 0)

        @pl.when(s < NDW)
        def _zero():
            def zchunk(j, carry):
                pltpu.sync_copy(zb, acc.at[pl.ds(s * RPT + j * ZB, ZB)])
                return carry

            lax.fori_loop(0, RPT // ZB, zchunk, 0)

        plsc.subcore_barrier()

        base0 = wid * EPW

        def stage(st, carry):
            pltpu.sync_copy(dst_hbm.at[wid, st], idx)
            cb = base0 + st * SCH * CHUNK

            def issue(li, p):
                pltpu.async_copy(y_hbm.at[pl.ds(cb + li * CHUNK, CHUNK)], yb[p], sy[p])

            def work(li, p):
                pltpu.make_async_copy(y_hbm.at[pl.ds(0, CHUNK)], yb[p], sy[p]).wait()
                pltpu.async_copy(yb[p], acc.at[idx.at[li]], sc_sem[p], add=True)

            def refill(li, p):
                pltpu.make_async_copy(yb[p], acc.at[pl.ds(0, CHUNK)], sc_sem[p]).wait()
                issue(li, p)

            for p in range(NBUF):
                issue(p, p)

            def group(k, carry2):
                i0 = NBUF * k
                for p in range(NBUF):
                    work(i0 + p, p)

                    @pl.when(i0 + p + NBUF < SCH)
                    def _r(p=p, i=i0 + p + NBUF):
                        refill(i, p)

                return carry2

            lax.fori_loop(0, SCH // NBUF, group, 0)
            for p in range(NBUF):
                pltpu.make_async_copy(yb[p], acc.at[pl.ds(0, CHUNK)], sc_sem[p]).wait()
            return carry

        lax.fori_loop(0, NST, stage, 0)
        plsc.subcore_barrier()

        @pl.when(s < NDW)
        def _dump():
            pltpu.sync_copy(
                acc.at[pl.ds(s * RPT, RPT)], out_hbm.at[c, pl.ds(s * RPT, RPT)]
            )

    return body(y, dst3)


def _final_body(p_ref, x_ref, gamma_ref, beta_ref, out_ref):
    h = p_ref[0] + p_ref[1]
    N = h.shape[0]
    mu = jnp.sum(h, axis=0, keepdims=True) / N
    var = jnp.sum(h * h, axis=0, keepdims=True) / N - mu * mu
    hn = (h - mu) * lax.rsqrt(var + EPS) * gamma_ref[...] + beta_ref[...]
    out_ref[...] = jax.nn.softplus(x_ref[...] + hn)


def _final(partials, x, gamma, beta):
    N, F = x.shape
    return pl.pallas_call(
        _final_body,
        out_shape=jax.ShapeDtypeStruct((N, F), jnp.float32),
    )(partials, x, gamma, beta)


def kernel(node_feats, edge_feats, edge_index, W_src, b_src, W_dst, b_dst,
           W_edge, b_edge, gamma_m, beta_m, gamma, beta):
    N, F = node_feats.shape
    E = edge_feats.shape[0]
    EPW = E // NW
    NCH = EPW // CHUNK
    src2 = edge_index[0].reshape(NW, EPW)
    dst2 = edge_index[1].reshape(NW, EPW)
    SCH = 50 if NCH % 50 == 0 else NCH
    dst3 = edge_index[1].reshape(NW, NCH // SCH, SCH, CHUNK)

    h_src, h_dst = _project(
        node_feats, W_src.T, b_src.reshape(1, -1), W_dst.T, b_dst.reshape(1, -1)
    )
    ga, gb = _gather_pair(h_src, h_dst, src2, dst2)
    stats = _edge_stats(ga, gb, edge_feats, W_edge.T, b_edge.reshape(1, -1))
    y = _edge_activate(
        ga, gb, edge_feats, W_edge.T, b_edge.reshape(1, -1), stats,
        gamma_m.reshape(1, -1), beta_m.reshape(1, -1),
    )
    partials = _scatter_sum(y, dst3, N)
    out = _final(partials, node_feats, gamma.reshape(1, -1), beta.reshape(1, -1))
    return out
